# baseline (speedup 1.0000x reference)
import functools

import jax
import jax.numpy as jnp
from jax import lax
from jax.experimental import pallas as pl
from jax.experimental.pallas import tpu as pltpu
from jax.experimental.pallas import tpu_sc as plsc

NUM_RELATIONS = 1000
D = 64
B = 16384
NC = 2
NS = 16
L = 16
NW = NC * NS
BPW = B // NW
NCHUNK = 4
CHUNK = BPW // NCHUNK

_mesh = plsc.VectorSubcoreMesh(core_axis_name="c", subcore_axis_name="s")


@functools.partial(
    pl.kernel,
    mesh=_mesh,
    out_type=jax.ShapeDtypeStruct((B,), jnp.float32),
    compiler_params=pltpu.CompilerParams(
        needs_layout_passes=False, use_tc_tiling_on_sc=False),
    scratch_types=[
        pltpu.VMEM((BPW,), jnp.int32),
        pltpu.VMEM((BPW, D), jnp.float32),
        pltpu.VMEM((BPW, D), jnp.float32),
        pltpu.VMEM((BPW, D), jnp.float32),
        pltpu.VMEM((L * L,), jnp.float32),
        pltpu.VMEM((BPW,), jnp.float32),
    ] + [pltpu.SemaphoreType.DMA] * NCHUNK,
)
def _distmult_sc(head_hbm, tail_hbm, idx_hbm, table_hbm, out_hbm,
                 idx_v, rel_v, head_v, tail_v, tbuf, out_v, *sems):
    wid = lax.axis_index("s") * NC + lax.axis_index("c")
    base = wid * BPW

    pltpu.sync_copy(idx_hbm.at[pl.ds(base, BPW)], idx_v)
    copies = []
    for c in range(NCHUNK):
        r0 = c * CHUNK
        copies.append((
            pltpu.async_copy(table_hbm.at[idx_v.at[pl.ds(r0, CHUNK)]],
                             rel_v.at[pl.ds(r0, CHUNK)], sems[c]),
            pltpu.async_copy(head_hbm.at[pl.ds(base + r0, CHUNK)],
                             head_v.at[pl.ds(r0, CHUNK)], sems[c]),
            pltpu.async_copy(tail_hbm.at[pl.ds(base + r0, CHUNK)],
                             tail_v.at[pl.ds(r0, CHUNK)], sems[c]),
        ))

    lane_iota_l = lax.iota(jnp.int32, L) * L

    def group_body(g, carry):
        row0 = g * L
        for r in range(L):
            row = row0 + r
            acc = (head_v[row, pl.ds(0, L)] * rel_v[row, pl.ds(0, L)]
                   * tail_v[row, pl.ds(0, L)])
            for q in range(1, D // L):
                acc = acc + (head_v[row, pl.ds(q * L, L)]
                             * rel_v[row, pl.ds(q * L, L)]
                             * tail_v[row, pl.ds(q * L, L)])
            tbuf[pl.ds(r * L, L)] = acc
        sums = plsc.load_gather(tbuf, [lane_iota_l])
        for l in range(1, L):
            sums = sums + plsc.load_gather(tbuf, [lane_iota_l + l])
        out_v[pl.ds(row0, L)] = sums
        return carry

    for c in range(NCHUNK):
        for cp in copies[c]:
            cp.wait()
        lax.fori_loop(c * (CHUNK // L), (c + 1) * (CHUNK // L), group_body, 0)

    pltpu.sync_copy(out_v, out_hbm.at[pl.ds(base, BPW)])


def kernel(head_emb, tail_emb, rel_idx, relation_embeddings):
    idx = rel_idx.astype(jnp.int32)
    return _distmult_sc(head_emb, tail_emb, idx, relation_embeddings)


# speedup vs baseline: 1.2572x; 1.2572x over previous
import functools

import jax
import jax.numpy as jnp
from jax import lax
from jax.experimental import pallas as pl
from jax.experimental.pallas import tpu as pltpu
from jax.experimental.pallas import tpu_sc as plsc

NUM_RELATIONS = 1000
D = 64
DP = 128
B = 16384
NC = 2
NS = 16
L = 16
NW = NC * NS
BPW = B // NW
NCHUNK = 4
CHUNK = BPW // NCHUNK
NBUF = 2

_mesh = plsc.VectorSubcoreMesh(core_axis_name="c", subcore_axis_name="s")


@functools.partial(
    pl.kernel,
    mesh=_mesh,
    out_type=jax.ShapeDtypeStruct((B,), jnp.float32),
    compiler_params=pltpu.CompilerParams(needs_layout_passes=False),
    scratch_types=[
        pltpu.VMEM((BPW,), jnp.int32),
        pltpu.VMEM((NBUF, CHUNK, DP), jnp.float32),
        pltpu.VMEM((NBUF, CHUNK, D), jnp.float32),
        pltpu.VMEM((NBUF, CHUNK, D), jnp.float32),
        pltpu.VMEM((L * L,), jnp.float32),
        pltpu.VMEM((BPW,), jnp.float32),
    ] + [pltpu.SemaphoreType.DMA] * NCHUNK,
)
def _distmult_sc(head_hbm, tail_hbm, idx_hbm, table_hbm, out_hbm,
                 idx_v, rel_v, head_v, tail_v, tbuf, out_v, *sems):
    wid = lax.axis_index("s") * NC + lax.axis_index("c")
    base = wid * BPW

    pltpu.sync_copy(idx_hbm.at[pl.ds(base, BPW)], idx_v)

    def issue(c):
        r0 = c * CHUNK
        slot = c % NBUF
        return (
            pltpu.async_copy(table_hbm.at[idx_v.at[pl.ds(r0, CHUNK)]],
                             rel_v.at[slot], sems[c]),
            pltpu.async_copy(head_hbm.at[pl.ds(base + r0, CHUNK)],
                             head_v.at[slot], sems[c]),
            pltpu.async_copy(tail_hbm.at[pl.ds(base + r0, CHUNK)],
                             tail_v.at[slot], sems[c]),
        )

    lane_iota_l = lax.iota(jnp.int32, L) * L
    copies = {0: issue(0), 1: issue(1)}

    for c in range(NCHUNK):
        slot = c % NBUF
        for cp in copies[c]:
            cp.wait()

        def group_body(g, carry, slot=slot, c=c):
            row0 = g * L
            for r in range(L):
                row = row0 + r
                acc = (head_v[slot, row, pl.ds(0, L)]
                       * rel_v[slot, row, pl.ds(0, L)]
                       * tail_v[slot, row, pl.ds(0, L)])
                for q in range(1, D // L):
                    acc = acc + (head_v[slot, row, pl.ds(q * L, L)]
                                 * rel_v[slot, row, pl.ds(q * L, L)]
                                 * tail_v[slot, row, pl.ds(q * L, L)])
                tbuf[pl.ds(r * L, L)] = acc
            sums = plsc.load_gather(tbuf, [lane_iota_l])
            for l in range(1, L):
                sums = sums + plsc.load_gather(tbuf, [lane_iota_l + l])
            out_v[pl.ds(c * CHUNK + row0, L)] = sums
            return carry

        lax.fori_loop(0, CHUNK // L, group_body, 0)
        if c + NBUF < NCHUNK:
            copies[c + NBUF] = issue(c + NBUF)

    pltpu.sync_copy(out_v, out_hbm.at[pl.ds(base, BPW)])


def kernel(head_emb, tail_emb, rel_idx, relation_embeddings):
    idx = rel_idx.astype(jnp.int32)
    table = jnp.pad(relation_embeddings, ((0, 0), (0, DP - D)))
    return _distmult_sc(head_emb, tail_emb, idx, table)


# speedup vs baseline: 1.3804x; 1.0980x over previous
import functools

import jax
import jax.numpy as jnp
from jax import lax
from jax.experimental import pallas as pl
from jax.experimental.pallas import tpu as pltpu
from jax.experimental.pallas import tpu_sc as plsc

NUM_RELATIONS = 1000
D = 64
B = 16384
NC = 2
NS = 16
L = 16
NW = NC * NS
BPW = B // NW
NCHUNK = 4
CB = BPW // NCHUNK
NBUF = 2

_mesh = plsc.VectorSubcoreMesh(core_axis_name="c", subcore_axis_name="s")


@functools.partial(
    pl.kernel,
    mesh=_mesh,
    out_type=jax.ShapeDtypeStruct((B,), jnp.float32),
    compiler_params=pltpu.CompilerParams(needs_layout_passes=False),
    scratch_types=[
        pltpu.VMEM((BPW,), jnp.int32),
        pltpu.VMEM((D, NUM_RELATIONS), jnp.float32),
        pltpu.VMEM((NBUF, D, CB), jnp.float32),
        pltpu.VMEM((NBUF, D, CB), jnp.float32),
        pltpu.VMEM((BPW,), jnp.float32),
        pltpu.SemaphoreType.DMA,
    ] + [pltpu.SemaphoreType.DMA] * NCHUNK,
)
def _distmult_sc(head_hbm, tail_hbm, idx_hbm, table_hbm, out_hbm,
                 idx_v, table_v, head_v, tail_v, out_v, sem0, *sems):
    wid = lax.axis_index("s") * NC + lax.axis_index("c")
    base = wid * BPW

    tbl_cp = pltpu.async_copy(table_hbm, table_v, sem0)
    idx_cp = pltpu.async_copy(idx_hbm.at[pl.ds(base, BPW)], idx_v, sem0)

    def issue(c):
        b0 = base + c * CB
        slot = c % NBUF
        return (
            pltpu.async_copy(head_hbm.at[:, pl.ds(b0, CB)],
                             head_v.at[slot], sems[c]),
            pltpu.async_copy(tail_hbm.at[:, pl.ds(b0, CB)],
                             tail_v.at[slot], sems[c]),
        )

    copies = {0: issue(0), 1: issue(1)}
    tbl_cp.wait()
    idx_cp.wait()

    for c in range(NCHUNK):
        slot = c % NBUF
        for cp in copies[c]:
            cp.wait()

        def j_body(j, carry, slot=slot, c=c):
            b0 = c * CB + j * L
            idxv = idx_v[pl.ds(b0, L)]
            accs = [jnp.zeros((L,), jnp.float32) for _ in range(4)]
            for d in range(D):
                rv = plsc.load_gather(
                    table_v, [jnp.full((L,), d, jnp.int32), idxv])
                hv = head_v[slot, d, pl.ds(j * L, L)]
                tv = tail_v[slot, d, pl.ds(j * L, L)]
                accs[d % 4] = accs[d % 4] + hv * rv * tv
            out_v[pl.ds(b0, L)] = (accs[0] + accs[1]) + (accs[2] + accs[3])
            return carry

        lax.fori_loop(0, CB // L, j_body, 0)
        if c + NBUF < NCHUNK:
            copies[c + NBUF] = issue(c + NBUF)

    pltpu.sync_copy(out_v, out_hbm.at[pl.ds(base, BPW)])


def kernel(head_emb, tail_emb, rel_idx, relation_embeddings):
    idx = rel_idx.astype(jnp.int32)
    return _distmult_sc(head_emb.T, tail_emb.T, idx, relation_embeddings.T)
